# baseline (speedup 1.0000x reference)
import dataclasses
import functools

import jax
import jax.numpy as jnp
from jax import lax
from jax.experimental import pallas as pl
from jax.experimental.pallas import tpu as pltpu
from jax.experimental.pallas import tpu_sc as plsc

V = 16
D = 16
NC = 2
NS = 16
NW = NC * NS
CHUNK = 3200


def _table_body(emb_ref, w_ref, b_ref, out_ref):
    out_ref[...] = lax.dot_general(
        emb_ref[...], w_ref[...],
        dimension_numbers=(((1,), (1,)), ((), ())),
        preferred_element_type=jnp.float32,
    ) + b_ref[...]


def _build_table(emb, W, b):
    b2 = jnp.broadcast_to(b[None, :], (V, V))
    return pl.pallas_call(
        _table_body,
        out_shape=jax.ShapeDtypeStruct((V, V), jnp.float32),
    )(emb, W, b2)


def _sc_compiler_params():
    cp = pltpu.CompilerParams(use_tc_tiling_on_sc=True)
    if "needs_layout_passes" in pltpu.CompilerParams.__dataclass_fields__:
        cp = dataclasses.replace(cp, needs_layout_passes=False)
    return cp


@functools.lru_cache(maxsize=None)
def _make_sc_gather(n_rows: int):
    assert n_rows % (NW * CHUNK) == 0
    per_w = n_rows // NW
    steps = per_w // CHUNK
    assert steps % 2 == 0
    groups = CHUNK // 16
    mesh = plsc.VectorSubcoreMesh(core_axis_name="c", subcore_axis_name="s")

    @functools.partial(
        pl.kernel,
        out_type=jax.ShapeDtypeStruct((n_rows * D,), jnp.float32),
        mesh=mesh,
        compiler_params=_sc_compiler_params(),
        scratch_types=[
            pltpu.VMEM((V * D,), jnp.float32),
            pltpu.VMEM((CHUNK,), jnp.int32),
            pltpu.VMEM((CHUNK,), jnp.int32),
            pltpu.VMEM((CHUNK * D,), jnp.float32),
            pltpu.VMEM((CHUNK * D,), jnp.float32),
            pltpu.SemaphoreType.DMA,
            pltpu.SemaphoreType.DMA,
        ],
    )
    def sc_gather(table_hbm, idx_hbm, out_hbm, table_v,
                  idx_v0, idx_v1, out_v0, out_v1, sem_in, sem_out):
        idx_bufs = (idx_v0, idx_v1)
        out_bufs = (out_v0, out_v1)
        wid = lax.axis_index("s") * NC + lax.axis_index("c")
        row0 = pl.multiple_of(wid * per_w, CHUNK)
        out_iota = lax.iota(jnp.int32, 16) * D

        pltpu.sync_copy(table_hbm, table_v)
        for b in range(2):
            pltpu.async_copy(
                idx_hbm.at[pl.ds(pl.multiple_of(row0 + b * CHUNK, CHUNK), CHUNK)],
                idx_bufs[b], sem_in)

        @pl.loop(0, steps, step=2)
        def _(s0):
            for b in range(2):
                s = s0 + b
                idx_v = idx_bufs[b]
                out_v = out_bufs[b]
                pltpu.make_async_copy(
                    idx_hbm.at[pl.ds(0, CHUNK)], idx_v, sem_in).wait()
                @pl.when(s0 >= 2)
                def _():
                    pltpu.make_async_copy(
                        out_v, out_hbm.at[pl.ds(0, CHUNK * D)], sem_out).wait()

                @plsc.parallel_loop(0, groups, unroll=4)
                def _(g):
                    ids = idx_v[pl.ds(g * 16, 16)]
                    in_base = ids * D
                    out_base = out_iota + g * (16 * D)
                    for c in range(D):
                        vals = plsc.load_gather(table_v, [in_base + c])
                        plsc.store_scatter(out_v, [out_base + c], vals)

                pltpu.async_copy(
                    out_v,
                    out_hbm.at[pl.ds(pl.multiple_of((row0 + s * CHUNK) * D, CHUNK * D),
                                     CHUNK * D)],
                    sem_out)

                @pl.when(s + 2 < steps)
                def _():
                    pltpu.async_copy(
                        idx_hbm.at[pl.ds(pl.multiple_of(row0, CHUNK) + (s + 2) * CHUNK,
                                         CHUNK)],
                        idx_v, sem_in)

        for b in range(2):
            pltpu.make_async_copy(
                out_bufs[b], out_hbm.at[pl.ds(0, CHUNK * D)], sem_out).wait()

    return sc_gather


def kernel(input_ids, emb, W, b):
    batch, seq = input_ids.shape
    n = batch * seq
    ids = input_ids.reshape(n).astype(jnp.int32)
    table = _build_table(emb, W, b).reshape(V * D)
    out = _make_sc_gather(n)(table, ids)
    return out.reshape(batch, seq, V)


# speedup vs baseline: 1.8052x; 1.8052x over previous
import functools

import jax
import jax.numpy as jnp
from jax import lax
from jax.experimental import pallas as pl

V = 16
D = 16
CH = 10240


def _table_body(emb_ref, w_ref, b_ref, out_ref):
    out_ref[...] = lax.dot_general(
        emb_ref[...], w_ref[...],
        dimension_numbers=(((1,), (1,)), ((), ())),
        preferred_element_type=jnp.float32,
    ) + b_ref[...]


def _build_table(emb, W, b):
    b2 = jnp.broadcast_to(b[None, :], (V, V))
    return pl.pallas_call(
        _table_body,
        out_shape=jax.ShapeDtypeStruct((V, V), jnp.float32),
    )(emb, W, b2)


def _onehot_body(ids_ref, table_ref, o_ref):
    ids = ids_ref[...]
    oh = (jnp.broadcast_to(ids[None, :], (V, CH))
          == lax.broadcasted_iota(jnp.int32, (V, CH), 0)).astype(jnp.float32)
    o_ref[...] = lax.dot_general(
        oh, table_ref[...],
        dimension_numbers=(((0,), (0,)), ((), ())),
        preferred_element_type=jnp.float32,
    )


@functools.lru_cache(maxsize=None)
def _make_lookup(n_rows: int):
    assert n_rows % CH == 0
    return pl.pallas_call(
        _onehot_body,
        grid=(n_rows // CH,),
        in_specs=[
            pl.BlockSpec((CH,), lambda i: (i,)),
            pl.BlockSpec((V, V), lambda i: (0, 0)),
        ],
        out_specs=pl.BlockSpec((CH, V), lambda i: (i, 0)),
        out_shape=jax.ShapeDtypeStruct((n_rows, V), jnp.float32),
    )


def kernel(input_ids, emb, W, b):
    batch, seq = input_ids.shape
    n = batch * seq
    ids = input_ids.reshape(n).astype(jnp.int32)
    table = _build_table(emb, W, b)
    out = _make_lookup(n)(ids, table)
    return out.reshape(batch, seq, V)
